# baseline (speedup 1.0000x reference)
import dataclasses
import functools

import jax
import jax.numpy as jnp
from jax import lax
from jax.experimental import pallas as pl
from jax.experimental.pallas import tpu as pltpu
from jax.experimental.pallas import tpu_sc as plsc

NC = 2
NS = 16
L = 16
NW = NC * NS
CHUNK = 128

BATCH = 4096
UNITS = 16
ROWS_PER_CORE = BATCH // NC
ROWS_PER_TILE = ROWS_PER_CORE // NS


def _body(bounds_hbm, col_hbm, row_hbm, val_hbm, table_hbm, bias_hbm, out_hbm,
          bounds_v, colv, rowv, valv, gath, gath_t, outbuf, biasv, acc_shared,
          gsem):
    c = lax.axis_index("c")
    s = lax.axis_index("s")
    wid = c * NS + s
    nnz = col_hbm.shape[0]
    iota = lax.iota(jnp.int32, L)
    zero16 = jnp.zeros((L,), jnp.float32)

    pltpu.sync_copy(bounds_hbm, bounds_v)
    s_w = jnp.max(plsc.load_gather(bounds_v, [jnp.full((L,), wid, jnp.int32)]))
    e_w = jnp.max(plsc.load_gather(bounds_v, [jnp.full((L,), wid + NW, jnp.int32)]))

    pltpu.sync_copy(bias_hbm, biasv)

    for t in range(ROWS_PER_TILE):
        outbuf[t, :] = zero16
    pltpu.sync_copy(outbuf, acc_shared.at[pl.ds(s * ROWS_PER_TILE, ROWS_PER_TILE)])
    plsc.subcore_barrier()

    row_base = c * ROWS_PER_CORE
    n_chunks = (e_w - s_w + CHUNK - 1) // CHUNK

    def chunk_body(k, carry):
        lo = s_w + k * CHUNK
        base = pl.multiple_of(jnp.minimum(lo, nnz - CHUNK), 8)
        pltpu.sync_copy(col_hbm.at[pl.ds(base, CHUNK)], colv)
        pltpu.sync_copy(row_hbm.at[pl.ds(base, CHUNK)], rowv)
        pltpu.sync_copy(val_hbm.at[pl.ds(base, CHUNK)], valv)
        pltpu.async_copy(table_hbm.at[colv], gath_t, gsem).wait()
        for g in range(CHUNK // L):
            r16 = rowv[pl.ds(g * L, L)]
            v16 = valv[pl.ds(g * L, L)]
            jg = base + g * L + iota
            rloc = r16 - row_base
            ok = (jg >= lo) & (jg < e_w) & (rloc >= 0) & (rloc < ROWS_PER_CORE)
            valv[pl.ds(g * L, L)] = jnp.where(ok, v16, 0.0)
            rowv[pl.ds(g * L, L)] = jnp.clip(rloc, 0, ROWS_PER_CORE - 1)
        for g in range(CHUNK // L):
            v16 = valv[pl.ds(g * L, L)]
            for t in range(L):
                j = g * L + t
                scale = jnp.take(v16, jnp.full((L,), t, jnp.int32))
                gath[j, :] = gath_t[j, :] * scale
        pltpu.sync_copy(gath, acc_shared.at[rowv], add=True)
        return carry

    lax.fori_loop(0, n_chunks, chunk_body, 0)
    plsc.subcore_barrier()

    bvec = biasv[...]
    pltpu.sync_copy(acc_shared.at[pl.ds(s * ROWS_PER_TILE, ROWS_PER_TILE)], outbuf)
    for t in range(ROWS_PER_TILE):
        outbuf[t, :] = outbuf[t, :] + bvec
    out_base = row_base + s * ROWS_PER_TILE
    pltpu.sync_copy(outbuf, out_hbm.at[pl.ds(out_base, ROWS_PER_TILE)])


def _make_sc_call():
    cp = pltpu.CompilerParams(use_tc_tiling_on_sc=False)
    if "needs_layout_passes" in pltpu.CompilerParams.__dataclass_fields__:
        cp = dataclasses.replace(cp, needs_layout_passes=False)
    mesh = plsc.VectorSubcoreMesh(
        core_axis_name="c", subcore_axis_name="s", num_cores=NC, num_subcores=NS
    )
    return pl.kernel(
        _body,
        out_type=jax.ShapeDtypeStruct((BATCH, UNITS), jnp.float32),
        mesh=mesh,
        scratch_types=[
            pltpu.VMEM((2 * NW,), jnp.int32),
            pltpu.VMEM((CHUNK,), jnp.int32),
            pltpu.VMEM((CHUNK,), jnp.int32),
            pltpu.VMEM((CHUNK,), jnp.float32),
            pltpu.VMEM((CHUNK, L), jnp.float32),
            pltpu.VMEM((CHUNK, L), jnp.float32),
            pltpu.VMEM((ROWS_PER_TILE, L), jnp.float32),
            pltpu.VMEM((L,), jnp.float32),
            pltpu.VMEM_SHARED((ROWS_PER_CORE, L), jnp.float32),
            pltpu.SemaphoreType.DMA,
        ],
        compiler_params=cp,
        name="sparse_dense_matmul_sc",
    )


VOCAB = 1000000
NBLK = VOCAB // 128
TAIL = VOCAB - NBLK * 128
TAIL_C0 = NBLK * 128
BPW = 245
NPAIR = 123


def _relayout_body(tbl_hbm, dense_hbm, tin0, tin1, tout0, tout1, tin_tail,
                   sin0, sin1, sout0, sout1):
    c = lax.axis_index("c")
    s = lax.axis_index("s")
    wid = c * NS + s
    iota = lax.iota(jnp.int32, L)
    base = wid * BPW

    def blk_c0(i):
        b = jnp.minimum(base + i, NBLK - 1)
        return pl.multiple_of(b * 128, 128)

    def start_in(i, tin, sem):
        return pltpu.async_copy(tbl_hbm.at[:, pl.ds(blk_c0(i), 128)], tin, sem)

    def wait_in(tin, sem):
        pltpu.make_async_copy(tbl_hbm.at[:, pl.ds(0, 128)], tin, sem).wait()

    def wait_out(tout, sem):
        pltpu.make_async_copy(tout, dense_hbm.at[pl.ds(0, 128)], sem).wait()

    def xpose(tin, tout, width):
        for j in range(width):
            tout[j, :] = plsc.load_gather(tin, [iota, jnp.full((L,), j, jnp.int32)])

    start_in(0, tin0, sin0)
    start_in(1, tin1, sin1)

    def pair(k, carry):
        a = 2 * k
        wait_in(tin0, sin0)

        @pl.when(k > 0)
        def _():
            wait_out(tout0, sout0)

        xpose(tin0, tout0, 128)
        pltpu.async_copy(tout0, dense_hbm.at[pl.ds(blk_c0(a), 128)], sout0)
        start_in(a + 2, tin0, sin0)

        wait_in(tin1, sin1)

        @pl.when(k > 0)
        def _():
            wait_out(tout1, sout1)

        xpose(tin1, tout1, 128)
        pltpu.async_copy(tout1, dense_hbm.at[pl.ds(blk_c0(a + 1), 128)], sout1)
        start_in(a + 3, tin1, sin1)
        return carry

    lax.fori_loop(0, NPAIR, pair, 0)
    wait_in(tin0, sin0)
    wait_in(tin1, sin1)
    wait_out(tout0, sout0)
    wait_out(tout1, sout1)

    @pl.when(wid == NW - 1)
    def _():
        pltpu.sync_copy(tbl_hbm.at[:, pl.ds(TAIL_C0, TAIL)], tin_tail)
        for j in range(TAIL):
            tout0[j, :] = plsc.load_gather(
                tin_tail, [iota, jnp.full((L,), j, jnp.int32)]
            )
        pltpu.sync_copy(tout0.at[pl.ds(0, TAIL)], dense_hbm.at[pl.ds(TAIL_C0, TAIL)])


def _make_relayout_call():
    cp = pltpu.CompilerParams(use_tc_tiling_on_sc=False)
    if "needs_layout_passes" in pltpu.CompilerParams.__dataclass_fields__:
        cp = dataclasses.replace(cp, needs_layout_passes=False)
    mesh = plsc.VectorSubcoreMesh(
        core_axis_name="c", subcore_axis_name="s", num_cores=NC, num_subcores=NS
    )
    return pl.kernel(
        _relayout_body,
        out_type=jax.ShapeDtypeStruct((VOCAB, UNITS), jnp.float32),
        mesh=mesh,
        scratch_types=[
            pltpu.VMEM((UNITS, 128), jnp.float32),
            pltpu.VMEM((UNITS, 128), jnp.float32),
            pltpu.VMEM((128, L), jnp.float32),
            pltpu.VMEM((128, L), jnp.float32),
            pltpu.VMEM((UNITS, TAIL), jnp.float32),
            pltpu.SemaphoreType.DMA,
            pltpu.SemaphoreType.DMA,
            pltpu.SemaphoreType.DMA,
            pltpu.SemaphoreType.DMA,
        ],
        compiler_params=cp,
        name="table_relayout_sc",
    )


def _align8(x):
    return (x // 8) * 8


def _worker_bounds(row_ids):
    nnz = row_ids.shape[0]
    s1 = jnp.sum((row_ids < ROWS_PER_CORE).astype(jnp.int32)).astype(jnp.int32)
    s1f = _align8(s1)
    s1c = _align8(s1 + 7)
    w = jnp.arange(NS, dtype=jnp.int32)
    starts0 = _align8(s1c * w // NS)
    ends0 = jnp.concatenate([starts0[1:], s1c[None]])
    starts1 = s1f + _align8((nnz - s1f) * w // NS)
    ends1 = jnp.concatenate([starts1[1:], jnp.full((1,), nnz, jnp.int32)])
    return jnp.concatenate([starts0, starts1, ends0, ends1])


_sc_call = None
_relayout_call = None


def kernel(row_ids, col_ids, values, kernel, bias):
    global _sc_call, _relayout_call
    if _sc_call is None:
        _sc_call = _make_sc_call()
        _relayout_call = _make_relayout_call()
    bounds = _worker_bounds(row_ids.astype(jnp.int32))
    dense = _relayout_call(kernel.T)
    return _sc_call(
        bounds,
        col_ids.astype(jnp.int32),
        row_ids.astype(jnp.int32),
        values,
        dense,
        bias,
    )


# speedup vs baseline: 3.5644x; 3.5644x over previous
import dataclasses
import functools

import jax
import jax.numpy as jnp
from jax import lax
from jax.experimental import pallas as pl
from jax.experimental.pallas import tpu as pltpu
from jax.experimental.pallas import tpu_sc as plsc

NC = 2
NS = 16
L = 16
NW = NC * NS
CHUNK = 128

BATCH = 4096
UNITS = 16
ROWS_PER_CORE = BATCH // NC
ROWS_PER_TILE = ROWS_PER_CORE // NS


def _body(bounds_hbm, col_hbm, row_hbm, val_hbm, table_hbm, bias_hbm, out_hbm,
          bounds_v, colv, rowv, valv, gath, gath_t, outbuf, biasv, acc_shared,
          gsem):
    c = lax.axis_index("c")
    s = lax.axis_index("s")
    wid = c * NS + s
    nnz = col_hbm.shape[0]
    iota = lax.iota(jnp.int32, L)
    zero16 = jnp.zeros((L,), jnp.float32)

    pltpu.sync_copy(bounds_hbm, bounds_v)
    s_w = jnp.max(plsc.load_gather(bounds_v, [jnp.full((L,), wid, jnp.int32)]))
    e_w = jnp.max(plsc.load_gather(bounds_v, [jnp.full((L,), wid + NW, jnp.int32)]))

    pltpu.sync_copy(bias_hbm, biasv)

    for t in range(ROWS_PER_TILE):
        outbuf[t, :] = zero16
    pltpu.sync_copy(outbuf, acc_shared.at[pl.ds(s * ROWS_PER_TILE, ROWS_PER_TILE)])
    plsc.subcore_barrier()

    row_base = c * ROWS_PER_CORE
    n_chunks = (e_w - s_w + CHUNK - 1) // CHUNK

    def chunk_body(k, carry):
        lo = s_w + k * CHUNK
        base = pl.multiple_of(jnp.minimum(lo, nnz - CHUNK), 8)
        pltpu.sync_copy(col_hbm.at[pl.ds(base, CHUNK)], colv)
        pltpu.sync_copy(row_hbm.at[pl.ds(base, CHUNK)], rowv)
        pltpu.sync_copy(val_hbm.at[pl.ds(base, CHUNK)], valv)
        pltpu.async_copy(table_hbm.at[colv], gath_t, gsem).wait()
        for g in range(CHUNK // L):
            r16 = rowv[pl.ds(g * L, L)]
            v16 = valv[pl.ds(g * L, L)]
            jg = base + g * L + iota
            rloc = r16 - row_base
            ok = (jg >= lo) & (jg < e_w) & (rloc >= 0) & (rloc < ROWS_PER_CORE)
            valv[pl.ds(g * L, L)] = jnp.where(ok, v16, 0.0)
            rowv[pl.ds(g * L, L)] = jnp.clip(rloc, 0, ROWS_PER_CORE - 1)
        for g in range(CHUNK // L):
            v16 = valv[pl.ds(g * L, L)]
            for t in range(L):
                j = g * L + t
                scale = jnp.take(v16, jnp.full((L,), t, jnp.int32))
                gath[j, :] = gath_t[j, :] * scale
        pltpu.sync_copy(gath, acc_shared.at[rowv], add=True)
        return carry

    lax.fori_loop(0, n_chunks, chunk_body, 0)
    plsc.subcore_barrier()

    bvec = biasv[...]
    pltpu.sync_copy(acc_shared.at[pl.ds(s * ROWS_PER_TILE, ROWS_PER_TILE)], outbuf)
    for t in range(ROWS_PER_TILE):
        outbuf[t, :] = outbuf[t, :] + bvec
    out_base = row_base + s * ROWS_PER_TILE
    pltpu.sync_copy(outbuf, out_hbm.at[pl.ds(out_base, ROWS_PER_TILE)])


def _make_sc_call():
    cp = pltpu.CompilerParams(use_tc_tiling_on_sc=False)
    if "needs_layout_passes" in pltpu.CompilerParams.__dataclass_fields__:
        cp = dataclasses.replace(cp, needs_layout_passes=False)
    mesh = plsc.VectorSubcoreMesh(
        core_axis_name="c", subcore_axis_name="s", num_cores=NC, num_subcores=NS
    )
    return pl.kernel(
        _body,
        out_type=jax.ShapeDtypeStruct((BATCH, UNITS), jnp.float32),
        mesh=mesh,
        scratch_types=[
            pltpu.VMEM((2 * NW,), jnp.int32),
            pltpu.VMEM((CHUNK,), jnp.int32),
            pltpu.VMEM((CHUNK,), jnp.int32),
            pltpu.VMEM((CHUNK,), jnp.float32),
            pltpu.VMEM((CHUNK, L), jnp.float32),
            pltpu.VMEM((CHUNK, L), jnp.float32),
            pltpu.VMEM((ROWS_PER_TILE, L), jnp.float32),
            pltpu.VMEM((L,), jnp.float32),
            pltpu.VMEM_SHARED((ROWS_PER_CORE, L), jnp.float32),
            pltpu.SemaphoreType.DMA,
        ],
        compiler_params=cp,
        name="sparse_dense_matmul_sc",
    )


VOCAB = 1000000
NBLK = VOCAB // 128
TAIL = VOCAB - NBLK * 128
TAIL_C0 = NBLK * 128
BPW = 245
NPAIR = 123


def _relayout_body(tbl_hbm, dense_hbm, tin0, tin1, tout0, tout1, tin_tail,
                   sin0, sin1, sout0, sout1):
    c = lax.axis_index("c")
    s = lax.axis_index("s")
    wid = c * NS + s
    iota = lax.iota(jnp.int32, L)
    base = wid * BPW

    def blk_c0(i):
        b = jnp.minimum(base + i, NBLK - 1)
        return pl.multiple_of(b * 128, 128)

    def start_in(i, tin, sem):
        return pltpu.async_copy(tbl_hbm.at[:, pl.ds(blk_c0(i), 128)], tin, sem)

    def wait_in(tin, sem):
        pltpu.make_async_copy(tbl_hbm.at[:, pl.ds(0, 128)], tin, sem).wait()

    def wait_out(tout, sem):
        pltpu.make_async_copy(tout, dense_hbm.at[pl.ds(0, 128 * UNITS)], sem).wait()

    def xpose(tin, tout, width):
        for j in range(width):
            tout[pl.ds(j * UNITS, UNITS)] = plsc.load_gather(
                tin, [iota, jnp.full((L,), j, jnp.int32)]
            )

    start_in(0, tin0, sin0)
    start_in(1, tin1, sin1)

    def pair(k, carry):
        a = 2 * k
        wait_in(tin0, sin0)

        @pl.when(k > 0)
        def _():
            wait_out(tout0, sout0)

        xpose(tin0, tout0, 128)
        pltpu.async_copy(
            tout0, dense_hbm.at[pl.ds(blk_c0(a) * UNITS, 128 * UNITS)], sout0
        )
        start_in(a + 2, tin0, sin0)

        wait_in(tin1, sin1)

        @pl.when(k > 0)
        def _():
            wait_out(tout1, sout1)

        xpose(tin1, tout1, 128)
        pltpu.async_copy(
            tout1, dense_hbm.at[pl.ds(blk_c0(a + 1) * UNITS, 128 * UNITS)], sout1
        )
        start_in(a + 3, tin1, sin1)
        return carry

    lax.fori_loop(0, NPAIR, pair, 0)
    wait_in(tin0, sin0)
    wait_in(tin1, sin1)
    wait_out(tout0, sout0)
    wait_out(tout1, sout1)

    @pl.when(wid == NW - 1)
    def _():
        pltpu.sync_copy(tbl_hbm.at[:, pl.ds(TAIL_C0, TAIL)], tin_tail)
        for j in range(TAIL):
            tout0[pl.ds(j * UNITS, UNITS)] = plsc.load_gather(
                tin_tail, [iota, jnp.full((L,), j, jnp.int32)]
            )
        pltpu.sync_copy(
            tout0.at[pl.ds(0, TAIL * UNITS)],
            dense_hbm.at[pl.ds(TAIL_C0 * UNITS, TAIL * UNITS)],
        )


def _make_relayout_call():
    cp = pltpu.CompilerParams(use_tc_tiling_on_sc=True)
    if "needs_layout_passes" in pltpu.CompilerParams.__dataclass_fields__:
        cp = dataclasses.replace(cp, needs_layout_passes=False)
    mesh = plsc.VectorSubcoreMesh(
        core_axis_name="c", subcore_axis_name="s", num_cores=NC, num_subcores=NS
    )
    return pl.kernel(
        _relayout_body,
        out_type=jax.ShapeDtypeStruct((VOCAB * UNITS,), jnp.float32),
        mesh=mesh,
        scratch_types=[
            pltpu.VMEM((UNITS, 128), jnp.float32),
            pltpu.VMEM((UNITS, 128), jnp.float32),
            pltpu.VMEM((128 * UNITS,), jnp.float32),
            pltpu.VMEM((128 * UNITS,), jnp.float32),
            pltpu.VMEM((UNITS, TAIL), jnp.float32),
            pltpu.SemaphoreType.DMA,
            pltpu.SemaphoreType.DMA,
            pltpu.SemaphoreType.DMA,
            pltpu.SemaphoreType.DMA,
        ],
        compiler_params=cp,
        name="table_relayout_sc",
    )


def _align8(x):
    return (x // 8) * 8


def _worker_bounds(row_ids):
    nnz = row_ids.shape[0]
    s1 = jnp.sum((row_ids < ROWS_PER_CORE).astype(jnp.int32)).astype(jnp.int32)
    s1f = _align8(s1)
    s1c = _align8(s1 + 7)
    w = jnp.arange(NS, dtype=jnp.int32)
    starts0 = _align8(s1c * w // NS)
    ends0 = jnp.concatenate([starts0[1:], s1c[None]])
    starts1 = s1f + _align8((nnz - s1f) * w // NS)
    ends1 = jnp.concatenate([starts1[1:], jnp.full((1,), nnz, jnp.int32)])
    return jnp.concatenate([starts0, starts1, ends0, ends1])


_sc_call = None
_relayout_call = None


def kernel(row_ids, col_ids, values, kernel, bias):
    global _sc_call, _relayout_call
    if _sc_call is None:
        _sc_call = _make_sc_call()
        _relayout_call = _make_relayout_call()
    bounds = _worker_bounds(row_ids.astype(jnp.int32))
    dense = _relayout_call(kernel.T).reshape(VOCAB, UNITS)
    return _sc_call(
        bounds,
        col_ids.astype(jnp.int32),
        row_ids.astype(jnp.int32),
        values,
        dense,
        bias,
    )
